# baseline (speedup 1.0000x reference)
import functools

import jax
import jax.numpy as jnp
from jax import lax
from jax.experimental import pallas as pl
from jax.experimental.pallas import tpu as pltpu
from jax.experimental.pallas import tpu_sc as plsc

_B = 16384
_D = 16
_C = 1000
_CPAD = 1024
_F = 8
_FD = _F * _D
_FDP = 128

_BM = 512
_NC = 2
_NS = 16
_NW = _NC * _NS
_BW = _B // _NW
_CHUNK = 128
_NCHUNK = _BW // _CHUNK


def _coarse_body(x_ref, ct_ref, out_ref):
    x = x_ref[...]
    ct = ct_ref[...]
    dots = lax.dot_general(x, ct, (((1,), (0,)), ((), ())),
                           preferred_element_type=jnp.float32)
    b2 = jnp.sum(ct * ct, axis=0)
    col = lax.broadcasted_iota(jnp.int32, (1, _CPAD), 1)
    pad_mask = jnp.where(col >= _C, jnp.float32(1e30), jnp.float32(0.0))
    scores = b2[None, :] - 2.0 * dots + pad_mask
    out_ref[0] = jnp.argmin(scores, axis=1).astype(jnp.int32).reshape(1, _BM)


_coarse_call = pl.pallas_call(
    _coarse_body,
    grid=(_B // _BM,),
    in_specs=[
        pl.BlockSpec((_BM, _D), lambda i: (i, 0)),
        pl.BlockSpec((_D, _CPAD), lambda i: (0, 0)),
    ],
    out_specs=pl.BlockSpec((1, 1, _BM), lambda i: (i, 0, 0)),
    out_shape=jax.ShapeDtypeStruct((_B // _BM, 1, _BM), jnp.int32),
)


def _sqrt16(x):
    i = plsc.bitcast(x, jnp.int32)
    i = jnp.int32(0x5F3759DF) - (i >> 1)
    y = plsc.bitcast(i, jnp.float32)
    half_x = 0.5 * x
    for _ in range(3):
        y = y * (1.5 - half_x * y * y)
    return x * y


@functools.cache
def _get_sc_fine():
    mesh = plsc.VectorSubcoreMesh(core_axis_name="c", subcore_axis_name="s",
                                  num_cores=_NC, num_subcores=_NS)
    return pl.kernel(
        _sc_fine_body,
        out_type=jax.ShapeDtypeStruct((_NW, 16), jnp.float32),
        mesh=mesh,
        scratch_types=[
            pltpu.VMEM((_NCHUNK, _CHUNK), jnp.int32),
            pltpu.VMEM((_D, _BW), jnp.float32),
            pltpu.VMEM((3, _CHUNK, _FDP), jnp.float32),
            pltpu.VMEM((16,), jnp.float32),
            pltpu.SemaphoreType.DMA,
            pltpu.SemaphoreType.DMA,
            pltpu.SemaphoreType.DMA,
        ],
        compiler_params=pltpu.CompilerParams(needs_layout_passes=False),
    )


def _sc_fine_body(feat_hbm, fine_hbm, idx_hbm, out_hbm,
                  idx_v, x_v, rows_v, acc_v, sem0, sem1, sem2):
    wid = lax.axis_index("s") * _NC + lax.axis_index("c")
    base = wid * _BW
    sems = (sem0, sem1, sem2)
    _NSLOT = 3

    pltpu.sync_copy(idx_hbm.at[wid], idx_v)
    pltpu.sync_copy(feat_hbm.at[:, pl.ds(base, _BW)], x_v)

    acc_v[...] = jnp.zeros((16,), jnp.float32)
    lanes = lax.iota(jnp.int32, 16)

    def gather_chunk(j):
        return pltpu.async_copy(fine_hbm.at[idx_v.at[j]], rows_v.at[j % _NSLOT],
                                sems[j % _NSLOT])

    def compute_chunk(j, cp):
        cp.wait()
        slot = j % _NSLOT

        def body(g, carry):
            r0 = g * 16
            row_idx = lanes + r0
            gbase = r0 + (j * _CHUNK)
            slot_idx = jnp.full((16,), slot, jnp.int32)
            zero = jnp.zeros((16,), jnp.float32)

            cmod = idx_v[j, pl.ds(r0, 16)] & (_D - 1)

            def dbody(d, accs):
                col_d = jnp.broadcast_to(d, (16,))
                xv = x_v[d, pl.ds(gbase, 16)]
                rot_d = (col_d + cmod) & (_D - 1)
                out = []
                for f in range(_F):
                    col = rot_d + (f * _D)
                    fv = plsc.load_gather(rows_v, [slot_idx, row_idx, col])
                    dd = xv - fv
                    out.append(accs[f] + dd * dd)
                return tuple(out)

            acc_f = lax.fori_loop(0, _D, dbody, (zero,) * _F)

            m = acc_f[0]
            for f in range(1, _F):
                m = jnp.minimum(m, acc_f[f])
            acc_v[...] = acc_v[...] + _sqrt16(m)
            return carry

        lax.fori_loop(0, _CHUNK // 16, body, 0)

    cps = [gather_chunk(j) for j in range(min(_NSLOT, _NCHUNK))]
    for j in range(_NCHUNK):
        compute_chunk(j, cps[j])
        if j + _NSLOT < _NCHUNK:
            cps.append(gather_chunk(j + _NSLOT))

    pltpu.sync_copy(acc_v, out_hbm.at[wid])


def kernel(features, coarse_prototypes, fine_prototypes):
    ct = jnp.zeros((_D, _CPAD), jnp.float32)
    ct = ct.at[:, :_C].set(coarse_prototypes.T)
    idx = _coarse_call(features, ct)
    idx3 = idx.reshape(_NW, _NCHUNK, _CHUNK)
    fine_r = fine_prototypes.reshape(_C, _F, _D)
    c_ids = jnp.arange(_C, dtype=jnp.int32)[:, None, None]
    e_ids = jnp.arange(_D, dtype=jnp.int32)[None, None, :]
    src_d = jnp.broadcast_to((e_ids - c_ids) % _D, (_C, _F, _D))
    fine_flat = jnp.take_along_axis(fine_r, src_d, axis=2).reshape(_C, _FD)
    feat_t = features.T
    parts = _get_sc_fine()(feat_t, fine_flat, idx3)
    return jnp.sum(parts) / jnp.float32(_B)


# speedup vs baseline: 1.3058x; 1.3058x over previous
import functools

import jax
import jax.numpy as jnp
from jax import lax
from jax.experimental import pallas as pl
from jax.experimental.pallas import tpu as pltpu
from jax.experimental.pallas import tpu_sc as plsc

_B = 16384
_D = 16
_C = 1000
_CPAD = 1024
_F = 8
_FD = _F * _D
_FDP = 128

_BM = 2048
_NC = 2
_NS = 16
_NW = _NC * _NS
_BW = _B // _NW
_CHUNK = 128
_NCHUNK = _BW // _CHUNK


def _coarse_body(cp_ref, xt_ref, out_ref):
    cp = cp_ref[...]
    xt = xt_ref[...]
    cp2 = -2.0 * cp
    dots2 = lax.dot_general(cp2, xt, (((1,), (0,)), ((), ())),
                            preferred_element_type=jnp.float32)
    row = lax.broadcasted_iota(jnp.int32, (_CPAD, 1), 0)
    b2p = jnp.sum(cp * cp, axis=1, keepdims=True) + jnp.where(
        row >= _C, jnp.float32(1e30), jnp.float32(0.0))
    a2 = jnp.sum(xt * xt, axis=0, keepdims=True)
    d2 = jnp.maximum((dots2 + a2) + b2p, 0.0)
    keys = (lax.bitcast_convert_type(d2, jnp.int32) & jnp.int32(~1023)) | row
    fkeys = lax.bitcast_convert_type(keys, jnp.float32)
    best = lax.bitcast_convert_type(jnp.min(fkeys, axis=0), jnp.int32)
    out_ref[0] = (best & jnp.int32(1023)).reshape(1, _BM)


_coarse_call = pl.pallas_call(
    _coarse_body,
    grid=(_B // _BM,),
    in_specs=[
        pl.BlockSpec((_CPAD, _D), lambda i: (0, 0)),
        pl.BlockSpec((_D, _BM), lambda i: (0, i)),
    ],
    out_specs=pl.BlockSpec((1, 1, _BM), lambda i: (i, 0, 0)),
    out_shape=jax.ShapeDtypeStruct((_B // _BM, 1, _BM), jnp.int32),
)


def _sqrt16(x):
    i = plsc.bitcast(x, jnp.int32)
    i = jnp.int32(0x5F3759DF) - (i >> 1)
    y = plsc.bitcast(i, jnp.float32)
    half_x = 0.5 * x
    for _ in range(3):
        y = y * (1.5 - half_x * y * y)
    return x * y


@functools.cache
def _get_sc_fine():
    mesh = plsc.VectorSubcoreMesh(core_axis_name="c", subcore_axis_name="s",
                                  num_cores=_NC, num_subcores=_NS)
    return pl.kernel(
        _sc_fine_body,
        out_type=jax.ShapeDtypeStruct((_NW, 16), jnp.float32),
        mesh=mesh,
        scratch_types=[
            pltpu.VMEM((_NCHUNK, _CHUNK), jnp.int32),
            pltpu.VMEM((_D, _BW), jnp.float32),
            pltpu.VMEM((3, _CHUNK, _FDP), jnp.float32),
            pltpu.VMEM((16,), jnp.float32),
            pltpu.SemaphoreType.DMA,
            pltpu.SemaphoreType.DMA,
            pltpu.SemaphoreType.DMA,
        ],
        compiler_params=pltpu.CompilerParams(needs_layout_passes=False),
    )


def _sc_fine_body(feat_hbm, fine_hbm, idx_hbm, out_hbm,
                  idx_v, x_v, rows_v, acc_v, sem0, sem1, sem2):
    wid = lax.axis_index("s") * _NC + lax.axis_index("c")
    base = wid * _BW
    sems = (sem0, sem1, sem2)
    _NSLOT = 3

    pltpu.sync_copy(idx_hbm.at[wid], idx_v)
    pltpu.sync_copy(feat_hbm.at[:, pl.ds(base, _BW)], x_v)

    acc_v[...] = jnp.zeros((16,), jnp.float32)
    lanes = lax.iota(jnp.int32, 16)

    def gather_chunk(j):
        return pltpu.async_copy(fine_hbm.at[idx_v.at[j]], rows_v.at[j % _NSLOT],
                                sems[j % _NSLOT])

    def compute_chunk(j, cp):
        cp.wait()
        slot = j % _NSLOT

        def body(g, carry):
            r0 = g * 16
            row_idx = lanes + r0
            gbase = r0 + (j * _CHUNK)
            slot_idx = jnp.full((16,), slot, jnp.int32)
            zero = jnp.zeros((16,), jnp.float32)

            cmod = idx_v[j, pl.ds(r0, 16)] & (_D - 1)

            def dbody(d, accs):
                col_d = jnp.broadcast_to(d, (16,))
                xv = x_v[d, pl.ds(gbase, 16)]
                rot_d = (col_d + cmod) & (_D - 1)
                out = []
                for f in range(_F):
                    col = rot_d + (f * _D)
                    fv = plsc.load_gather(rows_v, [slot_idx, row_idx, col])
                    dd = xv - fv
                    out.append(accs[f] + dd * dd)
                return tuple(out)

            acc_f = lax.fori_loop(0, _D, dbody, (zero,) * _F)

            m = acc_f[0]
            for f in range(1, _F):
                m = jnp.minimum(m, acc_f[f])
            acc_v[...] = acc_v[...] + _sqrt16(m)
            return carry

        lax.fori_loop(0, _CHUNK // 16, body, 0)

    cps = [gather_chunk(j) for j in range(min(_NSLOT, _NCHUNK))]
    for j in range(_NCHUNK):
        compute_chunk(j, cps[j])
        if j + _NSLOT < _NCHUNK:
            cps.append(gather_chunk(j + _NSLOT))

    pltpu.sync_copy(acc_v, out_hbm.at[wid])


def kernel(features, coarse_prototypes, fine_prototypes):
    cp_pad = jnp.zeros((_CPAD, _D), jnp.float32)
    cp_pad = cp_pad.at[:_C].set(coarse_prototypes)
    feat_t = features.T
    idx = _coarse_call(cp_pad, feat_t)
    idx3 = idx.reshape(_NW, _NCHUNK, _CHUNK)
    fine_r = fine_prototypes.reshape(_C, _F, _D)
    c_ids = jnp.arange(_C, dtype=jnp.int32)[:, None, None]
    e_ids = jnp.arange(_D, dtype=jnp.int32)[None, None, :]
    src_d = jnp.broadcast_to((e_ids - c_ids) % _D, (_C, _F, _D))
    fine_flat = jnp.take_along_axis(fine_r, src_d, axis=2).reshape(_C, _FD)
    parts = _get_sc_fine()(feat_t, fine_flat, idx3)
    return jnp.sum(parts) / jnp.float32(_B)
